# baseline (speedup 1.0000x reference)
import jax
import jax.numpy as jnp
from jax.experimental import pallas as pl

LAT_H = 48
LAT_W = 80
POOL_H = 8
POOL_W = 16
VIS_LEN = 30720
N_HEADS = 12
HEAD_DIM = 64
SPARSITY = 0.9

_ROW = N_HEADS * HEAD_DIM
_PARTS = VIS_LEN // (LAT_W * POOL_H)
_B = POOL_H
_C = LAT_W // POOL_W
_CHUNK = POOL_W
_S = _PARTS * _C
_N = _S * _S
_KCNT = int((1.0 - (1.0 - SPARSITY)) * _N)


def _permute_pool_body(q_ref, k_ref, v_ref, qr_ref, kr_ref, vr_ref,
                       qp_ref, kp_ref):
    qr_ref[...] = q_ref[...]
    kr_ref[...] = k_ref[...]
    vr_ref[...] = v_ref[...]
    b = pl.program_id(2)
    qs = jnp.sum(q_ref[...], axis=1, keepdims=True)
    ks = jnp.sum(k_ref[...], axis=1, keepdims=True)

    @pl.when(b == 0)
    def _init():
        qp_ref[...] = qs
        kp_ref[...] = ks

    @pl.when(b != 0)
    def _acc():
        qp_ref[...] += qs
        kp_ref[...] += ks


def _attn_mask_body(qp_ref, kp_ref, m_ref):
    qh = qp_ref[0]
    kh = kp_ref[0]
    s = jax.lax.dot_general(qh, kh, (((1,), (1,)), ((), ())),
                            preferred_element_type=jnp.float32)
    s = s * jnp.float32(2.0 ** -17)
    mx = jnp.max(s, axis=-1, keepdims=True)
    e = jnp.exp(s - mx)
    attn = e / jnp.sum(e, axis=-1, keepdims=True)
    bits = jax.lax.bitcast_convert_type(attn, jnp.int32)

    def body(i, ans):
        bit = jnp.int32(30) - i
        cand = ans | jax.lax.shift_left(jnp.int32(1), bit)
        cnt = jnp.sum((bits < cand).astype(jnp.int32))
        return jnp.where(cnt < _KCNT, cand, ans)

    ans = jax.lax.fori_loop(0, 31, body, jnp.int32(0))
    thr = jax.lax.bitcast_convert_type(ans, jnp.float32)
    m_ref[0] = (attn >= thr).astype(jnp.int8)


def kernel(q, k, v, cu_seqlens_q, cu_seqlens_kv, max_seqlen_q, max_seqlen_kv):
    L, H, D = q.shape
    nrows = L // _CHUNK
    q2 = q.reshape(nrows, _CHUNK, _ROW)
    k2 = k.reshape(nrows, _CHUNK, _ROW)
    v2 = v.reshape(nrows, _CHUNK, _ROW)

    rows_per_part = _B * _C
    in_spec = pl.BlockSpec(
        (1, _CHUNK, _ROW),
        lambda p, c, b: (p * rows_per_part + b * _C + c, 0, 0))
    out_spec = pl.BlockSpec(
        (1, _CHUNK, _ROW),
        lambda p, c, b: (p * rows_per_part + c * _B + b, 0, 0))
    pool_spec = pl.BlockSpec((1, 1, _ROW), lambda p, c, b: (p * _C + c, 0, 0))

    row_t = jax.ShapeDtypeStruct((nrows, _CHUNK, _ROW), jnp.float32)
    pool_t = jax.ShapeDtypeStruct((_S, 1, _ROW), jnp.float32)
    qr, kr, vr, qp, kp = pl.pallas_call(
        _permute_pool_body,
        grid=(_PARTS, _C, _B),
        in_specs=[in_spec, in_spec, in_spec],
        out_specs=[out_spec, out_spec, out_spec, pool_spec, pool_spec],
        out_shape=[row_t, row_t, row_t, pool_t, pool_t],
    )(q2, k2, v2)

    q_r = qr.reshape(L, H, D)
    k_r = kr.reshape(L, H, D)
    v_r = vr.reshape(L, H, D)

    qp_h = qp.reshape(_S, H, D).transpose(1, 0, 2)
    kp_h = kp.reshape(_S, H, D).transpose(1, 0, 2)

    head_spec = pl.BlockSpec((1, _S, D), lambda h: (h, 0, 0))
    mask_i8 = pl.pallas_call(
        _attn_mask_body,
        grid=(H,),
        in_specs=[head_spec, head_spec],
        out_specs=pl.BlockSpec((1, _S, _S), lambda h: (h, 0, 0)),
        out_shape=jax.ShapeDtypeStruct((H, _S, _S), jnp.int8),
    )(qp_h, kp_h)

    mask = mask_i8.astype(bool).reshape(1, H, _S, _S)
    return (q_r, k_r, v_r, mask)


# speedup vs baseline: 1.0139x; 1.0139x over previous
import jax
import jax.numpy as jnp
from jax.experimental import pallas as pl

LAT_H = 48
LAT_W = 80
POOL_H = 8
POOL_W = 16
VIS_LEN = 30720
N_HEADS = 12
HEAD_DIM = 64
SPARSITY = 0.9

_PARTS = VIS_LEN // (LAT_W * POOL_H)
_B = POOL_H
_C = LAT_W // POOL_W
_CHUNK = POOL_W
_RPP = _B * _C
_S = _PARTS * _C
_N = _S * _S
_KCNT = int((1.0 - (1.0 - SPARSITY)) * _N)


def _permute_pool_body(q_ref, k_ref, v_ref, qr_ref, kr_ref, vr_ref,
                       qp_ref, kp_ref):
    qr_ref[...] = q_ref[...]
    kr_ref[...] = k_ref[...]
    vr_ref[...] = v_ref[...]
    b = pl.program_id(2)
    qs = jnp.sum(q_ref[...], axis=0, keepdims=True)
    ks = jnp.sum(k_ref[...], axis=0, keepdims=True)

    @pl.when(b == 0)
    def _init():
        qp_ref[...] = qs
        kp_ref[...] = ks

    @pl.when(b != 0)
    def _acc():
        qp_ref[...] += qs
        kp_ref[...] += ks


def _attn_mask_body(qp_ref, kp_ref, m_ref):
    qh = qp_ref[0]
    kh = kp_ref[0]
    s = jax.lax.dot_general(qh, kh, (((1,), (1,)), ((), ())),
                            preferred_element_type=jnp.float32)
    s = s * jnp.float32(2.0 ** -17)
    mx = jnp.max(s, axis=-1, keepdims=True)
    e = jnp.exp(s - mx)
    attn = e / jnp.sum(e, axis=-1, keepdims=True)
    bits = jax.lax.bitcast_convert_type(attn, jnp.int32)

    def body(i, ans):
        bit = jnp.int32(30) - i
        cand = ans | jax.lax.shift_left(jnp.int32(1), bit)
        cnt = jnp.sum((bits < cand).astype(jnp.int32))
        return jnp.where(cnt < _KCNT, cand, ans)

    ans = jax.lax.fori_loop(0, 31, body, jnp.int32(0))
    thr = jax.lax.bitcast_convert_type(ans, jnp.float32)
    m_ref[0] = (attn >= thr).astype(jnp.int8)


def kernel(q, k, v, cu_seqlens_q, cu_seqlens_kv, max_seqlen_q, max_seqlen_kv):
    L, H, D = q.shape

    in_spec = pl.BlockSpec(
        (_CHUNK, H, D), lambda p, c, b: (p * _RPP + b * _C + c, 0, 0))
    out_spec = pl.BlockSpec(
        (_CHUNK, H, D), lambda p, c, b: (p * _RPP + c * _B + b, 0, 0))
    pool_spec = pl.BlockSpec((1, H, D), lambda p, c, b: (p * _C + c, 0, 0))

    row_t = jax.ShapeDtypeStruct((L, H, D), jnp.float32)
    pool_t = jax.ShapeDtypeStruct((_S, H, D), jnp.float32)
    q_r, k_r, v_r, qp, kp = pl.pallas_call(
        _permute_pool_body,
        grid=(_PARTS, _C, _B),
        in_specs=[in_spec, in_spec, in_spec],
        out_specs=[out_spec, out_spec, out_spec, pool_spec, pool_spec],
        out_shape=[row_t, row_t, row_t, pool_t, pool_t],
    )(q, k, v)

    qp_h = qp.transpose(1, 0, 2)
    kp_h = kp.transpose(1, 0, 2)

    head_spec = pl.BlockSpec((1, _S, D), lambda h: (h, 0, 0))
    mask_i8 = pl.pallas_call(
        _attn_mask_body,
        grid=(H,),
        in_specs=[head_spec, head_spec],
        out_specs=pl.BlockSpec((1, _S, _S), lambda h: (h, 0, 0)),
        out_shape=jax.ShapeDtypeStruct((H, _S, _S), jnp.int8),
    )(qp_h, kp_h)

    mask = mask_i8.astype(bool).reshape(1, H, _S, _S)
    return (q_r, k_r, v_r, mask)


# speedup vs baseline: 2.7548x; 2.7172x over previous
import jax
import jax.numpy as jnp
from jax.experimental import pallas as pl
from jax.experimental.pallas import tpu as pltpu

LAT_H = 48
LAT_W = 80
POOL_H = 8
POOL_W = 16
VIS_LEN = 30720
N_HEADS = 12
HEAD_DIM = 64
SPARSITY = 0.9

_PARTS = VIS_LEN // (LAT_W * POOL_H)
_B = POOL_H
_C = LAT_W // POOL_W
_CHUNK = POOL_W
_S = _PARTS * _C
_N = _S * _S
_KCNT = int((1.0 - (1.0 - SPARSITY)) * _N)


def _permute_pool_body(q_ref, k_ref, v_ref, qr_ref, kr_ref, vr_ref,
                       qp_ref, kp_ref, sem):
    p = pl.program_id(0)
    copies = []
    for src, dst in ((q_ref, qr_ref), (k_ref, kr_ref), (v_ref, vr_ref)):
        for c in range(5):
            cp = pltpu.make_async_copy(src.at[0, :, c], dst.at[p, c], sem)
            cp.start()
            copies.append(cp)
    qp_ref[...] = jnp.sum(q_ref[...], axis=(1, 3))
    kp_ref[...] = jnp.sum(k_ref[...], axis=(1, 3))
    for cp in copies:
        cp.wait()


def _attn_mask_body(qp_ref, kp_ref, m_ref):
    qh = qp_ref[0]
    kh = kp_ref[0]
    s = jax.lax.dot_general(qh, kh, (((1,), (1,)), ((), ())),
                            preferred_element_type=jnp.float32)
    s = s * jnp.float32(2.0 ** -17)
    mx = jnp.max(s, axis=-1, keepdims=True)
    e = jnp.exp(s - mx)
    attn = e / jnp.sum(e, axis=-1, keepdims=True)
    bits = jax.lax.bitcast_convert_type(attn, jnp.int32)

    def body(i, ans):
        bit = jnp.int32(30) - i
        cand = ans | jax.lax.shift_left(jnp.int32(1), bit)
        cnt = jnp.sum((bits < cand).astype(jnp.int32))
        return jnp.where(cnt < _KCNT, cand, ans)

    ans = jax.lax.fori_loop(0, 31, body, jnp.int32(0))
    thr = jax.lax.bitcast_convert_type(ans, jnp.float32)
    m_ref[0] = (attn >= thr).astype(jnp.int8)


def kernel(q, k, v, cu_seqlens_q, cu_seqlens_kv, max_seqlen_q, max_seqlen_kv):
    L, H, D = q.shape

    q6 = q.reshape(_PARTS, _B, _C, _CHUNK, H, D)
    k6 = k.reshape(_PARTS, _B, _C, _CHUNK, H, D)
    v6 = v.reshape(_PARTS, _B, _C, _CHUNK, H, D)

    in_spec = pl.BlockSpec((1, _B, _C, _CHUNK, H, D),
                           lambda p: (p, 0, 0, 0, 0, 0))
    any_spec = pl.BlockSpec(memory_space=pl.ANY)
    pool_spec = pl.BlockSpec((1, _C, H, D), lambda p: (p, 0, 0, 0))

    out6_t = jax.ShapeDtypeStruct((_PARTS, _C, _B, _CHUNK, H, D), jnp.float32)
    pool_t = jax.ShapeDtypeStruct((_PARTS, _C, H, D), jnp.float32)
    qr6, kr6, vr6, qp, kp = pl.pallas_call(
        _permute_pool_body,
        grid=(_PARTS,),
        in_specs=[in_spec, in_spec, in_spec],
        out_specs=[any_spec, any_spec, any_spec, pool_spec, pool_spec],
        out_shape=[out6_t, out6_t, out6_t, pool_t, pool_t],
        scratch_shapes=[pltpu.SemaphoreType.DMA],
    )(q6, k6, v6)

    q_r = qr6.reshape(L, H, D)
    k_r = kr6.reshape(L, H, D)
    v_r = vr6.reshape(L, H, D)

    qp_h = qp.reshape(_S, H, D).transpose(1, 0, 2)
    kp_h = kp.reshape(_S, H, D).transpose(1, 0, 2)

    head_spec = pl.BlockSpec((1, _S, D), lambda h: (h, 0, 0))
    mask_i8 = pl.pallas_call(
        _attn_mask_body,
        grid=(H,),
        in_specs=[head_spec, head_spec],
        out_specs=pl.BlockSpec((1, _S, _S), lambda h: (h, 0, 0)),
        out_shape=jax.ShapeDtypeStruct((H, _S, _S), jnp.int8),
    )(qp_h, kp_h)

    mask = mask_i8.astype(bool).reshape(1, H, _S, _S)
    return (q_r, k_r, v_r, mask)
